# baseline (speedup 1.0000x reference)
import functools

import jax
import jax.numpy as jnp
from jax import lax
from jax.experimental import pallas as pl
from jax.experimental.pallas import tpu as pltpu
from jax.experimental.pallas import tpu_sc as plsc

B = 4096
T, D = 20, 64
NUM_ROWS = 100000
L = 16
NC, NS = 2, 16
NW = NC * NS
BPW = B // NW
C = 16
NCHUNK = BPW // C


def _build():
    mesh = plsc.VectorSubcoreMesh(core_axis_name="c", subcore_axis_name="s")

    @functools.partial(
        pl.kernel,
        mesh=mesh,
        out_type=jax.ShapeDtypeStruct((B, T, D), jnp.float32),
        scratch_types=[
            pltpu.VMEM((BPW,), jnp.int32),
            pltpu.VMEM((C, T, D), jnp.float32),
            pltpu.VMEM((C, T, D), jnp.float32),
            pltpu.SemaphoreType.DMA,
            pltpu.SemaphoreType.DMA,
        ],
    )
    def run(x_hbm, idx_hbm, tab_hbm, out_hbm, idx_v, rows_v, xv,
            gsem, xsem):
        wid = lax.axis_index("s") * NC + lax.axis_index("c")
        base = wid * BPW
        pltpu.sync_copy(idx_hbm.at[pl.ds(base, BPW)], idx_v)

        def chunk_body(c, carry):
            cb = base + c * C
            xcopy = pltpu.async_copy(x_hbm.at[pl.ds(cb, C)], xv, xsem)
            ivec = idx_v[pl.ds(c * C, C)]
            handles = []
            for j in range(C):
                i = ivec[j]
                handles.append(pltpu.async_copy(
                    tab_hbm.at[pl.ds(i, 1)], rows_v.at[pl.ds(j, 1)], gsem))
            xcopy.wait()
            for h in handles:
                h.wait()

            def add_row(r, carry2):
                for t in range(T):
                    for k in range(D // L):
                        s = pl.ds(k * L, L)
                        plsc.addupdate(xv.at[r, t, s], rows_v[r, t, s])
                return carry2

            lax.fori_loop(0, C, add_row, 0)
            pltpu.sync_copy(xv, out_hbm.at[pl.ds(cb, C)])
            return carry

        lax.fori_loop(0, NCHUNK, chunk_body, 0)

    return run


_sc_call = _build()


@jax.jit
def kernel(x, idx, prompts):
    return _sc_call(x, idx.astype(jnp.int32), prompts)
